# Initial kernel scaffold; baseline (speedup 1.0000x reference)
import functools

import jax
import jax.numpy as jnp
from jax.experimental import pallas as pl

N_NODES = 1024
N_NEIGH = 32
T = 8
B = 16
F_IN = 128
EMB = 128
HID = 128
NCLS = 10

ROW_BLK = 128
INV_DEG = 1.0 / (N_NEIGH + 1)


def _build_a_body(adj_ref, a_ref):
    row0 = pl.program_id(0) * ROW_BLK
    col = jax.lax.broadcasted_iota(jnp.int32, (ROW_BLK, N_NODES), 1)
    row = jax.lax.broadcasted_iota(jnp.int32, (ROW_BLK, N_NODES), 0) + row0
    acc = (col == row).astype(jnp.float32)
    for j in range(N_NEIGH):
        acc += (col == adj_ref[:, j][:, None]).astype(jnp.float32)
    a_ref[...] = acc


def _gcn_body(x_ref, a_ref, w1_ref, b1_ref, w2_ref, b2_ref, out_ref):
    x = x_ref[0, 0]
    y = jnp.dot(x, w1_ref[...], preferred_element_type=jnp.float32)
    z = jnp.dot(a_ref[...], y, preferred_element_type=jnp.float32)
    h1 = jnp.maximum(z * INV_DEG + b1_ref[...], 0.0)
    y2 = jnp.dot(h1, w2_ref[...], preferred_element_type=jnp.float32)
    z2 = jnp.dot(a_ref[...], y2, preferred_element_type=jnp.float32)
    h2 = jnp.maximum(z2 * INV_DEG + b2_ref[...], 0.0)
    out_ref[0, 0] = jnp.sum(h2, axis=0)


def _lstm_body(seq_ref, wih_ref, whh_ref, b_ref, wc_ref, bc_ref, out_ref):
    h = jnp.zeros((B, HID), dtype=jnp.float32)
    c = jnp.zeros((B, HID), dtype=jnp.float32)
    for t in range(T):
        x = seq_ref[t]
        gates = (jnp.dot(x, wih_ref[...], preferred_element_type=jnp.float32)
                 + jnp.dot(h, whh_ref[...], preferred_element_type=jnp.float32)
                 + b_ref[...])
        i = jax.nn.sigmoid(gates[:, 0 * HID:1 * HID])
        f = jax.nn.sigmoid(gates[:, 1 * HID:2 * HID])
        g = jnp.tanh(gates[:, 2 * HID:3 * HID])
        o = jax.nn.sigmoid(gates[:, 3 * HID:4 * HID])
        c = f * c + i * g
        h = o * jnp.tanh(c)
    hr = jnp.maximum(h, 0.0)
    logits = jnp.dot(hr, wc_ref[...], preferred_element_type=jnp.float32) + bc_ref[...]
    logits = logits - jnp.max(logits, axis=1, keepdims=True)
    e = jnp.exp(logits)
    out_ref[...] = e / jnp.sum(e, axis=1, keepdims=True)


@jax.jit
def kernel(node_features, adjacent_mappings, W1, b1, W2, b2, W_ih, W_hh, b_ih, b_hh, Wc, bc):
    adj = adjacent_mappings.astype(jnp.int32)

    a_mat = pl.pallas_call(
        _build_a_body,
        grid=(N_NODES // ROW_BLK,),
        in_specs=[pl.BlockSpec((ROW_BLK, N_NEIGH), lambda i: (i, 0))],
        out_specs=pl.BlockSpec((ROW_BLK, N_NODES), lambda i: (i, 0)),
        out_shape=jax.ShapeDtypeStruct((N_NODES, N_NODES), jnp.float32),
    )(adj)

    seq = pl.pallas_call(
        _gcn_body,
        grid=(T, B),
        in_specs=[
            pl.BlockSpec((1, 1, N_NODES, F_IN), lambda t, b: (t, b, 0, 0)),
            pl.BlockSpec((N_NODES, N_NODES), lambda t, b: (0, 0)),
            pl.BlockSpec((F_IN, F_IN), lambda t, b: (0, 0)),
            pl.BlockSpec((1, F_IN), lambda t, b: (0, 0)),
            pl.BlockSpec((F_IN, EMB), lambda t, b: (0, 0)),
            pl.BlockSpec((1, EMB), lambda t, b: (0, 0)),
        ],
        out_specs=pl.BlockSpec((1, 1, EMB), lambda t, b: (t, b, 0)),
        out_shape=jax.ShapeDtypeStruct((T, B, EMB), jnp.float32),
    )(node_features, a_mat, W1, b1.reshape(1, F_IN), W2, b2.reshape(1, EMB))

    out = pl.pallas_call(
        _lstm_body,
        in_specs=[
            pl.BlockSpec((T, B, EMB), lambda: (0, 0, 0)),
            pl.BlockSpec((EMB, 4 * HID), lambda: (0, 0)),
            pl.BlockSpec((HID, 4 * HID), lambda: (0, 0)),
            pl.BlockSpec((1, 4 * HID), lambda: (0, 0)),
            pl.BlockSpec((HID, NCLS), lambda: (0, 0)),
            pl.BlockSpec((1, NCLS), lambda: (0, 0)),
        ],
        out_specs=pl.BlockSpec((B, NCLS), lambda: (0, 0)),
        out_shape=jax.ShapeDtypeStruct((B, NCLS), jnp.float32),
    )(seq, W_ih.T, W_hh.T, (b_ih + b_hh).reshape(1, 4 * HID), Wc.T, bc.reshape(1, NCLS))

    return out


# speedup vs baseline: 147.4752x; 147.4752x over previous
import functools

import jax
import jax.numpy as jnp
from jax.experimental import pallas as pl

N_NODES = 1024
N_NEIGH = 32
T = 8
B = 16
F_IN = 128
EMB = 128
HID = 128
NCLS = 10

ROW_BLK = 128
INV_DEG = 1.0 / (N_NEIGH + 1)


def _build_a_body(adj_ref, a_ref):
    row0 = pl.program_id(0) * ROW_BLK
    col = jax.lax.broadcasted_iota(jnp.int32, (ROW_BLK, N_NODES), 1)
    row = jax.lax.broadcasted_iota(jnp.int32, (ROW_BLK, N_NODES), 0) + row0
    acc = (col == row).astype(jnp.float32)
    for j in range(N_NEIGH):
        acc += (col == adj_ref[:, j][:, None]).astype(jnp.float32)
    a_ref[...] = acc


def _gcn_body(x_ref, a_ref, w1_ref, b1_ref, w2_ref, b2_ref, out_ref):
    x = x_ref[0, 0]
    y = jnp.dot(x, w1_ref[...], preferred_element_type=jnp.float32)
    z = jnp.dot(a_ref[...], y, preferred_element_type=jnp.float32)
    h1 = jnp.maximum(z * INV_DEG + b1_ref[...], 0.0)
    y2 = jnp.dot(h1, w2_ref[...], preferred_element_type=jnp.float32)
    z2 = jnp.dot(a_ref[...], y2, preferred_element_type=jnp.float32)
    h2 = jnp.maximum(z2 * INV_DEG + b2_ref[...], 0.0)
    out_ref[0, 0, 0] = jnp.sum(h2, axis=0)


def _lstm_body(seq_ref, wih_ref, whh_ref, b_ref, wc_ref, bc_ref, out_ref):
    h = jnp.zeros((B, HID), dtype=jnp.float32)
    c = jnp.zeros((B, HID), dtype=jnp.float32)
    for t in range(T):
        x = seq_ref[t]
        gates = (jnp.dot(x, wih_ref[...], preferred_element_type=jnp.float32)
                 + jnp.dot(h, whh_ref[...], preferred_element_type=jnp.float32)
                 + b_ref[...])
        i = jax.nn.sigmoid(gates[:, 0 * HID:1 * HID])
        f = jax.nn.sigmoid(gates[:, 1 * HID:2 * HID])
        g = jnp.tanh(gates[:, 2 * HID:3 * HID])
        o = jax.nn.sigmoid(gates[:, 3 * HID:4 * HID])
        c = f * c + i * g
        h = o * jnp.tanh(c)
    hr = jnp.maximum(h, 0.0)
    logits = jnp.dot(hr, wc_ref[...], preferred_element_type=jnp.float32) + bc_ref[...]
    logits = logits - jnp.max(logits, axis=1, keepdims=True)
    e = jnp.exp(logits)
    out_ref[...] = e / jnp.sum(e, axis=1, keepdims=True)


@jax.jit
def kernel(node_features, adjacent_mappings, W1, b1, W2, b2, W_ih, W_hh, b_ih, b_hh, Wc, bc):
    adj = adjacent_mappings.astype(jnp.int32)

    a_mat = pl.pallas_call(
        _build_a_body,
        grid=(N_NODES // ROW_BLK,),
        in_specs=[pl.BlockSpec((ROW_BLK, N_NEIGH), lambda i: (i, 0))],
        out_specs=pl.BlockSpec((ROW_BLK, N_NODES), lambda i: (i, 0)),
        out_shape=jax.ShapeDtypeStruct((N_NODES, N_NODES), jnp.float32),
    )(adj)

    seq = pl.pallas_call(
        _gcn_body,
        grid=(T, B),
        in_specs=[
            pl.BlockSpec((1, 1, N_NODES, F_IN), lambda t, b: (t, b, 0, 0)),
            pl.BlockSpec((N_NODES, N_NODES), lambda t, b: (0, 0)),
            pl.BlockSpec((F_IN, F_IN), lambda t, b: (0, 0)),
            pl.BlockSpec((1, F_IN), lambda t, b: (0, 0)),
            pl.BlockSpec((F_IN, EMB), lambda t, b: (0, 0)),
            pl.BlockSpec((1, EMB), lambda t, b: (0, 0)),
        ],
        out_specs=pl.BlockSpec((1, 1, 1, EMB), lambda t, b: (t, b, 0, 0)),
        out_shape=jax.ShapeDtypeStruct((T, B, 1, EMB), jnp.float32),
    )(node_features, a_mat, W1, b1.reshape(1, F_IN), W2, b2.reshape(1, EMB))
    seq = seq.reshape(T, B, EMB)

    out = pl.pallas_call(
        _lstm_body,
        in_specs=[
            pl.BlockSpec((T, B, EMB), lambda: (0, 0, 0)),
            pl.BlockSpec((EMB, 4 * HID), lambda: (0, 0)),
            pl.BlockSpec((HID, 4 * HID), lambda: (0, 0)),
            pl.BlockSpec((1, 4 * HID), lambda: (0, 0)),
            pl.BlockSpec((HID, NCLS), lambda: (0, 0)),
            pl.BlockSpec((1, NCLS), lambda: (0, 0)),
        ],
        out_specs=pl.BlockSpec((B, NCLS), lambda: (0, 0)),
        out_shape=jax.ShapeDtypeStruct((B, NCLS), jnp.float32),
    )(seq, W_ih.T, W_hh.T, (b_ih + b_hh).reshape(1, 4 * HID), Wc.T, bc.reshape(1, NCLS))

    return out
